# baseline (speedup 1.0000x reference)
import functools

import jax
import jax.numpy as jnp
from jax import lax
from jax.experimental import pallas as pl
from jax.experimental.pallas import tpu as pltpu
from jax.experimental.pallas import tpu_sc as plsc

VOCAB = 100000
EMBED = 100
EMBED_PAD = 128
BATCH = 4096
HIST = 50
HIST_PAD = 56

NC = 2
NS = 16
NW = NC * NS

B_PER_W = BATCH // NW
CHUNK_B = 2
CHUNK = CHUNK_B * HIST_PAD
CHUNK_PAD = 128
NCHUNK = B_PER_W // CHUNK_B
RING = 4


def _make_gather():
    mesh = plsc.VectorSubcoreMesh(core_axis_name="c", subcore_axis_name="s")

    @functools.partial(
        pl.kernel,
        mesh=mesh,
        out_type=jax.ShapeDtypeStruct((BATCH * HIST_PAD, EMBED_PAD), jnp.float32),
        scratch_types=[
            pltpu.VMEM((NCHUNK, CHUNK_PAD), jnp.int32),
            pltpu.VMEM((RING, CHUNK_PAD, EMBED_PAD), jnp.float32),
            pltpu.SemaphoreType.DMA((RING,)),
        ],
        compiler_params=pltpu.CompilerParams(use_tc_tiling_on_sc=True),
    )
    def gather_kernel(idx_hbm, table_hbm, out_hbm, idx_v, rows_v, sems):
        wid = lax.axis_index("s") * NC + lax.axis_index("c")
        base_row = wid * B_PER_W * HIST_PAD
        pltpu.sync_copy(idx_hbm.at[wid], idx_v)

        def fire(c, b):
            pltpu.async_copy(table_hbm.at[idx_v.at[c]], rows_v.at[b], sems.at[b])

        def drain(b):
            pltpu.make_async_copy(
                table_hbm.at[idx_v.at[0]], rows_v.at[b], sems.at[b]
            ).wait()

        for b in range(RING):
            fire(b, b)

        @pl.loop(0, NCHUNK // RING)
        def _group(i):
            c0 = i * RING
            for b in range(RING):
                c = c0 + b
                drain(b)
                row0 = pl.multiple_of(base_row + c * CHUNK, 8)
                pltpu.sync_copy(
                    rows_v.at[b].at[pl.ds(0, CHUNK)],
                    out_hbm.at[pl.ds(row0, CHUNK)],
                )

                @pl.when(c + RING < NCHUNK)
                def _():
                    fire(c + RING, b)

    return gather_kernel


_gather = _make_gather()


def kernel(x, glove_embd):
    idx = jnp.pad(x.astype(jnp.int32), ((0, 0), (0, HIST_PAD - HIST)))
    idx = idx.reshape(NW, NCHUNK, CHUNK)
    idx = jnp.pad(idx, ((0, 0), (0, 0), (0, CHUNK_PAD - CHUNK)))
    table = jnp.pad(glove_embd, ((0, 0), (0, EMBED_PAD - EMBED)))
    out = _gather(idx, table)
    return out.reshape(BATCH, HIST_PAD, EMBED_PAD)[:, :HIST, :EMBED]


# speedup vs baseline: 1.8575x; 1.8575x over previous
import functools

import jax
import jax.numpy as jnp
from jax import lax
from jax.experimental import pallas as pl
from jax.experimental.pallas import tpu as pltpu
from jax.experimental.pallas import tpu_sc as plsc

VOCAB = 100000
EMBED = 100
EMBED_PAD = 128
BATCH = 4096
HIST = 50
HIST_PAD = 56

NC = 2
NS = 16
NW = NC * NS

B = BATCH * HIST_PAD
B_PER_W = B // NW
CHUNK = 128
NCHUNK = B_PER_W // CHUNK
RING = 4


def _make_gather():
    mesh = plsc.VectorSubcoreMesh(core_axis_name="c", subcore_axis_name="s")

    @functools.partial(
        pl.kernel,
        mesh=mesh,
        out_type=jax.ShapeDtypeStruct((B, EMBED_PAD), jnp.float32),
        scratch_types=[
            pltpu.VMEM((NCHUNK, CHUNK), jnp.int32),
            pltpu.VMEM((RING, CHUNK, EMBED_PAD), jnp.float32),
            pltpu.SemaphoreType.DMA((RING,)),
        ],
        compiler_params=pltpu.CompilerParams(use_tc_tiling_on_sc=True),
    )
    def gather_kernel(idx_hbm, table_hbm, out_hbm, idx_v, rows_v, sems):
        wid = lax.axis_index("s") * NC + lax.axis_index("c")
        base = wid * B_PER_W
        pltpu.sync_copy(idx_hbm.at[wid], idx_v)

        def fire(c, b):
            pltpu.async_copy(table_hbm.at[idx_v.at[c]], rows_v.at[b], sems.at[b])

        def drain(b):
            pltpu.make_async_copy(
                table_hbm.at[idx_v.at[0]], rows_v.at[b], sems.at[b]
            ).wait()

        for b in range(RING):
            fire(b, b)

        @pl.loop(0, NCHUNK // RING)
        def _group(i):
            c0 = i * RING
            for b in range(RING):
                c = c0 + b
                drain(b)
                off = pl.multiple_of(base + c * CHUNK, CHUNK)
                pltpu.sync_copy(rows_v.at[b], out_hbm.at[pl.ds(off, CHUNK)])

                @pl.when(c + RING < NCHUNK)
                def _():
                    fire(c + RING, b)

    return gather_kernel


_gather = _make_gather()


def kernel(x, glove_embd):
    idx = jnp.pad(x.astype(jnp.int32), ((0, 0), (0, HIST_PAD - HIST)))
    idx = idx.reshape(NW, NCHUNK, CHUNK)
    table = jnp.pad(glove_embd, ((0, 0), (0, EMBED_PAD - EMBED)))
    out = _gather(idx, table)
    return out.reshape(BATCH, HIST_PAD, EMBED_PAD)[:, :HIST, :EMBED]


# speedup vs baseline: 7.2872x; 3.9232x over previous
import functools

import jax
import jax.numpy as jnp
from jax import lax
from jax.experimental import pallas as pl
from jax.experimental.pallas import tpu as pltpu
from jax.experimental.pallas import tpu_sc as plsc

VOCAB = 100000
EMBED = 100
EMBED_PAD = 128
BATCH = 4096
HIST = 50
HIST_PAD = 56

NC = 2
NS = 16
NW = NC * NS

B = BATCH * HIST_PAD
B_PER_W = B // NW
CHUNK = 128
NCHUNK = B_PER_W // CHUNK
RING = 4


def _make_gather():
    mesh = plsc.VectorSubcoreMesh(core_axis_name="c", subcore_axis_name="s")

    @functools.partial(
        pl.kernel,
        mesh=mesh,
        out_type=jax.ShapeDtypeStruct((B, EMBED_PAD), jnp.float32),
        scratch_types=[
            pltpu.VMEM((NCHUNK, CHUNK), jnp.int32),
            pltpu.VMEM((RING, CHUNK, EMBED_PAD), jnp.float32),
            pltpu.SemaphoreType.DMA((RING,)),
        ],
        compiler_params=pltpu.CompilerParams(use_tc_tiling_on_sc=True),
    )
    def gather_kernel(idx_hbm, table_hbm, out_hbm, idx_v, rows_v, sems):
        wid = lax.axis_index("s") * NC + lax.axis_index("c")
        base = wid * B_PER_W
        pltpu.sync_copy(idx_hbm.at[wid], idx_v)

        def fire(c, b):
            pltpu.async_copy(table_hbm.at[idx_v.at[c]], rows_v.at[b], sems.at[b])

        def drain(b):
            pltpu.make_async_copy(
                table_hbm.at[idx_v.at[0]], rows_v.at[b], sems.at[b]
            ).wait()

        for b in range(RING):
            fire(b, b)

        @pl.loop(0, NCHUNK // RING)
        def _group(i):
            c0 = i * RING
            for b in range(RING):
                c = c0 + b
                drain(b)
                off = pl.multiple_of(base + c * CHUNK, CHUNK)
                pltpu.sync_copy(rows_v.at[b], out_hbm.at[pl.ds(off, CHUNK)])

                @pl.when(c + RING < NCHUNK)
                def _():
                    fire(c + RING, b)

    return gather_kernel


_gather = _make_gather()


def kernel(x, glove_embd):
    dummy = (jnp.arange(BATCH * (HIST_PAD - HIST), dtype=jnp.int32)
             % VOCAB).reshape(BATCH, HIST_PAD - HIST)
    idx = jnp.concatenate([x.astype(jnp.int32), dummy], axis=1)
    idx = idx.reshape(NW, NCHUNK, CHUNK)
    table = jnp.pad(glove_embd, ((0, 0), (0, EMBED_PAD - EMBED)))
    out = _gather(idx, table)
    return out.reshape(BATCH, HIST_PAD, EMBED_PAD)[:, :HIST, :EMBED]


# speedup vs baseline: 11.1289x; 1.5272x over previous
import functools

import jax
import jax.numpy as jnp
from jax import lax
from jax.experimental import pallas as pl
from jax.experimental.pallas import tpu as pltpu
from jax.experimental.pallas import tpu_sc as plsc

VOCAB = 100000
EMBED = 100
EMBED_PAD = 128
BATCH = 4096
HIST = 50
HIST_PAD = 56

NC = 2
NS = 16
NW = NC * NS

B = BATCH * HIST_PAD
B_PER_W = B // NW
CHUNK = 128
NCHUNK = B_PER_W // CHUNK
RING = 4


def _make_gather():
    mesh = plsc.VectorSubcoreMesh(core_axis_name="c", subcore_axis_name="s")

    @functools.partial(
        pl.kernel,
        mesh=mesh,
        out_type=jax.ShapeDtypeStruct((B, EMBED_PAD), jnp.float32),
        scratch_types=[
            pltpu.VMEM((NCHUNK, CHUNK), jnp.int32),
            pltpu.VMEM((RING, CHUNK, EMBED_PAD), jnp.float32),
            pltpu.SemaphoreType.DMA((RING,)),
        ],
        compiler_params=pltpu.CompilerParams(use_tc_tiling_on_sc=True),
    )
    def gather_kernel(idx_hbm, table_hbm, out_hbm, idx_v, rows_v, sems):
        wid = lax.axis_index("s") * NC + lax.axis_index("c")
        base = wid * B_PER_W
        pltpu.sync_copy(idx_hbm.at[wid], idx_v)

        def fire(c, b):
            pltpu.async_copy(table_hbm.at[idx_v.at[c]], rows_v.at[b], sems.at[b])

        def drain(b):
            pltpu.make_async_copy(
                table_hbm.at[idx_v.at[0]], rows_v.at[b], sems.at[b]
            ).wait()

        for b in range(RING):
            fire(b, b)

        @pl.loop(0, NCHUNK // RING)
        def _group(i):
            c0 = i * RING
            for b in range(RING):
                c = c0 + b
                drain(b)
                off = pl.multiple_of(base + c * CHUNK, CHUNK)
                pltpu.sync_copy(rows_v.at[b], out_hbm.at[pl.ds(off, CHUNK)])

                @pl.when(c + RING < NCHUNK)
                def _():
                    fire(c + RING, b)

    return gather_kernel


_gather = _make_gather()

VBLK = 2048


def _transpose_pad_kernel(gt_ref, eye_ref, out_ref):
    res = jax.lax.dot_general(
        gt_ref[...], eye_ref[...], (((0,), (0,)), ((), ())),
        preferred_element_type=jnp.float32,
        precision=jax.lax.Precision.HIGHEST,
    )
    out_ref[...] = jnp.concatenate(
        [res, jnp.zeros((VBLK, EMBED_PAD - EMBED), jnp.float32)], axis=1
    )


def _transpose_pad(gt, eye):
    grid = (VOCAB + VBLK - 1) // VBLK
    return pl.pallas_call(
        _transpose_pad_kernel,
        grid=(grid,),
        in_specs=[
            pl.BlockSpec((EMBED, VBLK), lambda i: (0, i)),
            pl.BlockSpec((EMBED, EMBED), lambda i: (0, 0)),
        ],
        out_specs=pl.BlockSpec((VBLK, EMBED_PAD), lambda i: (i, 0)),
        out_shape=jax.ShapeDtypeStruct((VOCAB, EMBED_PAD), jnp.float32),
    )(gt, eye)


def kernel(x, glove_embd):
    dummy = (jnp.arange(BATCH * (HIST_PAD - HIST), dtype=jnp.int32)
             % VOCAB).reshape(BATCH, HIST_PAD - HIST)
    idx = jnp.concatenate([x.astype(jnp.int32), dummy], axis=1)
    idx = idx.reshape(NW, NCHUNK, CHUNK)
    table = _transpose_pad(glove_embd.T, jnp.eye(EMBED, dtype=jnp.float32))
    out = _gather(idx, table)
    return out.reshape(BATCH, HIST_PAD, EMBED_PAD)[:, :HIST, :EMBED]
